# Initial kernel scaffold; baseline (speedup 1.0000x reference)
import jax
import jax.numpy as jnp
from jax.experimental import pallas as pl


def kernel(plaintext, key, sbox, rcon):
    raise NotImplementedError("write your pallas kernel here")


# speedup vs baseline: 19.0841x; 19.0841x over previous
import functools

import jax
import jax.numpy as jnp
from jax import lax
from jax.experimental import pallas as pl
from jax.experimental.pallas import tpu as pltpu
from jax.experimental.pallas import tpu_sc as plsc

_MESH = plsc.VectorSubcoreMesh(core_axis_name="c", subcore_axis_name="s")


def _gat(ref, idx):
    return plsc.load_gather(ref, [idx])


@functools.partial(
    pl.kernel,
    mesh=_MESH,
    compiler_params=pltpu.CompilerParams(needs_layout_passes=False),
    out_type=jax.ShapeDtypeStruct((16,), jnp.int32),
    scratch_types=[
        pltpu.VMEM((256,), jnp.int32),
        pltpu.VMEM((16,), jnp.int32),
        pltpu.VMEM((16,), jnp.int32),
        pltpu.VMEM((160,), jnp.int32),
        pltpu.VMEM((16,), jnp.int32),
    ],
)
def _aes_sc(pt_hbm, key_hbm, sbox_hbm, rcon_hbm, out_hbm,
            sbox_v, pt_v, key_v, rcon_v, tmp_v):
    cid = lax.axis_index("c")
    sid = lax.axis_index("s")

    @pl.when(jnp.logical_and(cid == 0, sid == 0))
    def _():
        pltpu.sync_copy(sbox_hbm, sbox_v)
        pltpu.sync_copy(pt_hbm, pt_v)
        pltpu.sync_copy(key_hbm, key_v)
        pltpu.sync_copy(rcon_hbm, rcon_v)

        lane = lax.broadcasted_iota(jnp.int32, (16,), 0)
        mod4 = lane & 3
        base = lane - mod4
        shift_perm = (lane + (mod4 << 2)) & 15
        rot1 = base + ((mod4 + 1) & 3)
        rot2 = base + ((mod4 + 2) & 3)
        rot3 = base + ((mod4 + 3) & 3)
        temp_idx = ((mod4 + 1) & 3) + 12
        sh4_idx = jnp.maximum(lane - 4, 0)
        sh8_idx = jnp.maximum(lane - 8, 0)
        m4 = lane >= 4
        m8 = lane >= 8
        zero = jnp.zeros((16,), jnp.int32)

        rk = key_v[...]
        round_keys = [rk]
        for r in range(1, 11):
            tmp_v[...] = rk
            t = _gat(sbox_v, _gat(tmp_v, temp_idx))
            t = t ^ rcon_v[pl.ds(16 * (r - 1), 16)]
            g4 = _gat(tmp_v, sh4_idx)
            a = rk ^ jnp.where(m4, g4, zero)
            tmp_v[...] = a
            g8 = _gat(tmp_v, sh8_idx)
            rk = (a ^ jnp.where(m8, g8, zero)) ^ t
            round_keys.append(rk)

        state = pt_v[...] ^ round_keys[0]
        for r in range(1, 10):
            tmp_v[...] = state
            sb = _gat(sbox_v, _gat(tmp_v, shift_perm))
            tmp_v[...] = sb
            b1 = _gat(tmp_v, rot1)
            b2 = _gat(tmp_v, rot2)
            b3 = _gat(tmp_v, rot3)
            t = sb ^ b1 ^ b2 ^ b3
            x = sb ^ b1
            xt = ((x << 1) ^ ((x >> 7) & 1) * 27) & 255
            state = (sb ^ t ^ xt) ^ round_keys[r]
        tmp_v[...] = state
        sb = _gat(sbox_v, _gat(tmp_v, shift_perm))
        tmp_v[...] = sb ^ round_keys[10]
        pltpu.sync_copy(tmp_v, out_hbm)


def kernel(plaintext, key, sbox, rcon):
    mask = (jnp.arange(16) % 4 == 0).astype(jnp.int32)
    rcon_sched = (rcon.astype(jnp.int32)[:, None] * mask[None, :]).reshape(160)
    return _aes_sc(plaintext.astype(jnp.int32), key.astype(jnp.int32),
                   sbox.astype(jnp.int32), rcon_sched)


# speedup vs baseline: 20.1698x; 1.0569x over previous
import functools

import jax
import jax.numpy as jnp
from jax import lax
from jax.experimental import pallas as pl
from jax.experimental.pallas import tpu as pltpu
from jax.experimental.pallas import tpu_sc as plsc

_MESH = plsc.VectorSubcoreMesh(
    core_axis_name="c", subcore_axis_name="s", num_cores=1)


def _gat(ref, idx):
    return plsc.load_gather(ref, [idx])


@functools.partial(
    pl.kernel,
    mesh=_MESH,
    compiler_params=pltpu.CompilerParams(needs_layout_passes=False),
    out_type=jax.ShapeDtypeStruct((16,), jnp.int32),
    scratch_types=[
        pltpu.VMEM((256,), jnp.int32),
        pltpu.VMEM((16,), jnp.int32),
        pltpu.VMEM((16,), jnp.int32),
        pltpu.VMEM((160,), jnp.int32),
        pltpu.VMEM((16,), jnp.int32),
    ],
)
def _aes_sc(pt_hbm, key_hbm, sbox_hbm, rcon_hbm, out_hbm,
            sbox_v, pt_v, key_v, rcon_v, tmp_v):
    cid = lax.axis_index("c")
    sid = lax.axis_index("s")

    @pl.when(jnp.logical_and(cid == 0, sid == 0))
    def _():
        pltpu.sync_copy(sbox_hbm, sbox_v)
        pltpu.sync_copy(pt_hbm, pt_v)
        pltpu.sync_copy(key_hbm, key_v)
        pltpu.sync_copy(rcon_hbm, rcon_v)

        lane = lax.broadcasted_iota(jnp.int32, (16,), 0)
        mod4 = lane & 3
        base = lane - mod4
        shift_perm = (lane + (mod4 << 2)) & 15
        rot1 = base + ((mod4 + 1) & 3)
        rot2 = base + ((mod4 + 2) & 3)
        rot3 = base + ((mod4 + 3) & 3)
        temp_idx = ((mod4 + 1) & 3) + 12
        sh4_idx = jnp.maximum(lane - 4, 0)
        sh8_idx = jnp.maximum(lane - 8, 0)
        m4 = lane >= 4
        m8 = lane >= 8
        zero = jnp.zeros((16,), jnp.int32)

        rk = key_v[...]
        round_keys = [rk]
        for r in range(1, 11):
            tmp_v[...] = rk
            t = _gat(sbox_v, _gat(tmp_v, temp_idx))
            t = t ^ rcon_v[pl.ds(16 * (r - 1), 16)]
            g4 = _gat(tmp_v, sh4_idx)
            a = rk ^ jnp.where(m4, g4, zero)
            tmp_v[...] = a
            g8 = _gat(tmp_v, sh8_idx)
            rk = (a ^ jnp.where(m8, g8, zero)) ^ t
            round_keys.append(rk)

        state = pt_v[...] ^ round_keys[0]
        for r in range(1, 10):
            tmp_v[...] = state
            sb = _gat(sbox_v, _gat(tmp_v, shift_perm))
            tmp_v[...] = sb
            b1 = _gat(tmp_v, rot1)
            b2 = _gat(tmp_v, rot2)
            b3 = _gat(tmp_v, rot3)
            t = sb ^ b1 ^ b2 ^ b3
            x = sb ^ b1
            xt = ((x << 1) ^ ((x >> 7) & 1) * 27) & 255
            state = (sb ^ t ^ xt) ^ round_keys[r]
        tmp_v[...] = state
        sb = _gat(sbox_v, _gat(tmp_v, shift_perm))
        tmp_v[...] = sb ^ round_keys[10]
        pltpu.sync_copy(tmp_v, out_hbm)


def kernel(plaintext, key, sbox, rcon):
    mask = (jnp.arange(16) % 4 == 0).astype(jnp.int32)
    rcon_sched = (rcon.astype(jnp.int32)[:, None] * mask[None, :]).reshape(160)
    return _aes_sc(plaintext.astype(jnp.int32), key.astype(jnp.int32),
                   sbox.astype(jnp.int32), rcon_sched)


# speedup vs baseline: 21.5469x; 1.0683x over previous
import functools

import jax
import jax.numpy as jnp
from jax import lax
from jax.experimental import pallas as pl
from jax.experimental.pallas import tpu as pltpu
from jax.experimental.pallas import tpu_sc as plsc

_MESH = plsc.VectorSubcoreMesh(
    core_axis_name="c", subcore_axis_name="s", num_cores=1)


def _gat(ref, idx):
    return plsc.load_gather(ref, [idx])


@functools.partial(
    pl.kernel,
    mesh=_MESH,
    compiler_params=pltpu.CompilerParams(
        needs_layout_passes=False,
        disable_bounds_checks=True,
    ),
    out_type=jax.ShapeDtypeStruct((16,), jnp.int32),
    scratch_types=[
        pltpu.VMEM((256,), jnp.int32),
        pltpu.VMEM((16,), jnp.int32),
        pltpu.VMEM((16,), jnp.int32),
        pltpu.VMEM((160,), jnp.int32),
        pltpu.VMEM((16,), jnp.int32),
        pltpu.SemaphoreType.DMA,
    ],
)
def _aes_sc(pt_hbm, key_hbm, sbox_hbm, rcon_hbm, out_hbm,
            sbox_v, pt_v, key_v, rcon_v, tmp_v, sem):
    sid = lax.axis_index("s")

    @pl.when(sid == 0)
    def _():
        c1 = pltpu.async_copy(sbox_hbm, sbox_v, sem)
        c2 = pltpu.async_copy(pt_hbm, pt_v, sem)
        c3 = pltpu.async_copy(key_hbm, key_v, sem)
        c4 = pltpu.async_copy(rcon_hbm, rcon_v, sem)
        c1.wait()
        c2.wait()
        c3.wait()
        c4.wait()

        lane = lax.broadcasted_iota(jnp.int32, (16,), 0)
        mod4 = lane & 3
        base = lane - mod4
        shift_perm = (lane + (mod4 << 2)) & 15
        rot1 = base + ((mod4 + 1) & 3)
        rot2 = base + ((mod4 + 2) & 3)
        rot3 = base + ((mod4 + 3) & 3)
        temp_idx = ((mod4 + 1) & 3) + 12
        sh4_idx = jnp.maximum(lane - 4, 0)
        sh8_idx = jnp.maximum(lane - 8, 0)
        m4 = lane >= 4
        m8 = lane >= 8
        zero = jnp.zeros((16,), jnp.int32)

        rk = key_v[...]
        round_keys = [rk]
        for r in range(1, 11):
            tmp_v[...] = rk
            t = _gat(sbox_v, _gat(tmp_v, temp_idx))
            t = t ^ rcon_v[pl.ds(16 * (r - 1), 16)]
            g4 = _gat(tmp_v, sh4_idx)
            a = rk ^ jnp.where(m4, g4, zero)
            tmp_v[...] = a
            g8 = _gat(tmp_v, sh8_idx)
            rk = (a ^ jnp.where(m8, g8, zero)) ^ t
            round_keys.append(rk)

        state = pt_v[...] ^ round_keys[0]
        for r in range(1, 10):
            tmp_v[...] = state
            sb = _gat(sbox_v, _gat(tmp_v, shift_perm))
            tmp_v[...] = sb
            b1 = _gat(tmp_v, rot1)
            b2 = _gat(tmp_v, rot2)
            b3 = _gat(tmp_v, rot3)
            t = sb ^ b1 ^ b2 ^ b3
            x = sb ^ b1
            xt = ((x << 1) ^ ((x >> 7) & 1) * 27) & 255
            state = (sb ^ t ^ xt) ^ round_keys[r]
        tmp_v[...] = state
        sb = _gat(sbox_v, _gat(tmp_v, shift_perm))
        tmp_v[...] = sb ^ round_keys[10]
        pltpu.sync_copy(tmp_v, out_hbm)


def kernel(plaintext, key, sbox, rcon):
    mask = (jnp.arange(16) % 4 == 0).astype(jnp.int32)
    rcon_sched = (rcon.astype(jnp.int32)[:, None] * mask[None, :]).reshape(160)
    return _aes_sc(plaintext.astype(jnp.int32), key.astype(jnp.int32),
                   sbox.astype(jnp.int32), rcon_sched)


# speedup vs baseline: 21.8262x; 1.0130x over previous
import functools

import jax
import jax.numpy as jnp
from jax import lax
from jax.experimental import pallas as pl
from jax.experimental.pallas import tpu as pltpu
from jax.experimental.pallas import tpu_sc as plsc

_MESH = plsc.VectorSubcoreMesh(
    core_axis_name="c", subcore_axis_name="s", num_cores=1)

_SBOX_OFF = 0
_PT_OFF = 256
_KEY_OFF = 272
_RCON_OFF = 288
_PACKED = 448


def _gat(ref, idx):
    return plsc.load_gather(ref, [idx])


@functools.partial(
    pl.kernel,
    mesh=_MESH,
    compiler_params=pltpu.CompilerParams(
        needs_layout_passes=False,
        disable_bounds_checks=True,
    ),
    out_type=jax.ShapeDtypeStruct((16,), jnp.int32),
    scratch_types=[
        pltpu.VMEM((_PACKED,), jnp.int32),
        pltpu.VMEM((16,), jnp.int32),
        pltpu.SemaphoreType.DMA,
    ],
)
def _aes_sc(in_hbm, out_hbm, in_v, tmp_v, sem):
    sid = lax.axis_index("s")

    @pl.when(sid == 0)
    def _():
        pltpu.async_copy(in_hbm, in_v, sem).wait()

        lane = lax.broadcasted_iota(jnp.int32, (16,), 0)
        mod4 = lane & 3
        base = lane - mod4
        shift_perm = (lane + (mod4 << 2)) & 15
        rot1 = base + ((mod4 + 1) & 3)
        rot2 = base + ((mod4 + 2) & 3)
        rot3 = base + ((mod4 + 3) & 3)
        temp_idx = ((mod4 + 1) & 3) + 12
        sh4_idx = jnp.maximum(lane - 4, 0)
        sh8_idx = jnp.maximum(lane - 8, 0)
        m4 = lane >= 4
        m8 = lane >= 8
        zero = jnp.zeros((16,), jnp.int32)

        rk = in_v[pl.ds(_KEY_OFF, 16)]
        round_keys = [rk]
        for r in range(1, 11):
            tmp_v[...] = rk
            t = _gat(in_v, _gat(tmp_v, temp_idx))
            t = t ^ in_v[pl.ds(_RCON_OFF + 16 * (r - 1), 16)]
            g4 = _gat(tmp_v, sh4_idx)
            a = rk ^ jnp.where(m4, g4, zero)
            tmp_v[...] = a
            g8 = _gat(tmp_v, sh8_idx)
            rk = (a ^ jnp.where(m8, g8, zero)) ^ t
            round_keys.append(rk)

        state = in_v[pl.ds(_PT_OFF, 16)] ^ round_keys[0]
        for r in range(1, 10):
            tmp_v[...] = state
            sb = _gat(in_v, _gat(tmp_v, shift_perm))
            tmp_v[...] = sb
            b1 = _gat(tmp_v, rot1)
            b2 = _gat(tmp_v, rot2)
            b3 = _gat(tmp_v, rot3)
            t = sb ^ b1 ^ b2 ^ b3
            x = sb ^ b1
            xt = ((x << 1) ^ ((x >> 7) & 1) * 27) & 255
            state = (sb ^ t ^ xt) ^ round_keys[r]
        tmp_v[...] = state
        sb = _gat(in_v, _gat(tmp_v, shift_perm))
        tmp_v[...] = sb ^ round_keys[10]
        pltpu.sync_copy(tmp_v, out_hbm)


def kernel(plaintext, key, sbox, rcon):
    mask = (jnp.arange(16) % 4 == 0).astype(jnp.int32)
    rcon_sched = (rcon.astype(jnp.int32)[:, None] * mask[None, :]).reshape(160)
    packed = jnp.concatenate([
        sbox.astype(jnp.int32), plaintext.astype(jnp.int32),
        key.astype(jnp.int32), rcon_sched,
    ])
    return _aes_sc(packed)


# speedup vs baseline: 21.9125x; 1.0040x over previous
import functools

import jax
import jax.numpy as jnp
from jax import lax
from jax.experimental import pallas as pl
from jax.experimental.pallas import tpu as pltpu
from jax.experimental.pallas import tpu_sc as plsc

_MESH = plsc.VectorSubcoreMesh(
    core_axis_name="c", subcore_axis_name="s", num_cores=1)

_SBOX_OFF = 0
_PT_OFF = 256
_KEY_OFF = 272
_RCON_OFF = 288
_PACKED = 448


def _gat(ref, idx):
    return plsc.load_gather(ref, [idx])


@functools.partial(
    pl.kernel,
    mesh=_MESH,
    compiler_params=pltpu.CompilerParams(
        needs_layout_passes=False,
        disable_bounds_checks=True,
        skip_device_barrier=True,
    ),
    out_type=jax.ShapeDtypeStruct((16,), jnp.int32),
    scratch_types=[
        pltpu.VMEM((_PACKED,), jnp.int32),
        pltpu.VMEM((16,), jnp.int32),
        pltpu.SemaphoreType.DMA,
    ],
)
def _aes_sc(in_hbm, out_hbm, in_v, tmp_v, sem):
    sid = lax.axis_index("s")

    @pl.when(sid == 0)
    def _():
        pltpu.async_copy(in_hbm, in_v, sem).wait()

        lane = lax.broadcasted_iota(jnp.int32, (16,), 0)
        mod4 = lane & 3
        base = lane - mod4
        shift_perm = (lane + (mod4 << 2)) & 15
        rot1 = base + ((mod4 + 1) & 3)
        rot2 = base + ((mod4 + 2) & 3)
        rot3 = base + ((mod4 + 3) & 3)
        temp_idx = ((mod4 + 1) & 3) + 12
        sh4_idx = jnp.maximum(lane - 4, 0)
        sh8_idx = jnp.maximum(lane - 8, 0)
        m4 = lane >= 4
        m8 = lane >= 8
        zero = jnp.zeros((16,), jnp.int32)

        rk = in_v[pl.ds(_KEY_OFF, 16)]
        round_keys = [rk]
        for r in range(1, 11):
            tmp_v[...] = rk
            t = _gat(in_v, _gat(tmp_v, temp_idx))
            t = t ^ in_v[pl.ds(_RCON_OFF + 16 * (r - 1), 16)]
            g4 = _gat(tmp_v, sh4_idx)
            a = rk ^ jnp.where(m4, g4, zero)
            tmp_v[...] = a
            g8 = _gat(tmp_v, sh8_idx)
            rk = (a ^ jnp.where(m8, g8, zero)) ^ t
            round_keys.append(rk)

        state = in_v[pl.ds(_PT_OFF, 16)] ^ round_keys[0]
        for r in range(1, 10):
            tmp_v[...] = state
            sb = _gat(in_v, _gat(tmp_v, shift_perm))
            tmp_v[...] = sb
            b1 = _gat(tmp_v, rot1)
            b2 = _gat(tmp_v, rot2)
            b3 = _gat(tmp_v, rot3)
            t = sb ^ b1 ^ b2 ^ b3
            x = sb ^ b1
            xt = ((x << 1) ^ ((x >> 7) & 1) * 27) & 255
            state = (sb ^ t ^ xt) ^ round_keys[r]
        tmp_v[...] = state
        sb = _gat(in_v, _gat(tmp_v, shift_perm))
        tmp_v[...] = sb ^ round_keys[10]
        pltpu.sync_copy(tmp_v, out_hbm)


def kernel(plaintext, key, sbox, rcon):
    mask = (jnp.arange(16) % 4 == 0).astype(jnp.int32)
    rcon_sched = (rcon.astype(jnp.int32)[:, None] * mask[None, :]).reshape(160)
    packed = jnp.concatenate([
        sbox.astype(jnp.int32), plaintext.astype(jnp.int32),
        key.astype(jnp.int32), rcon_sched,
    ])
    return _aes_sc(packed)
